# baseline (speedup 1.0000x reference)
import functools

import jax
import jax.numpy as jnp
from jax import lax
from jax.experimental import pallas as pl
from jax.experimental.pallas import tpu as pltpu
from jax.experimental.pallas import tpu_sc as plsc

N_NODES = 10000
N_EDGES = 320000
D_FEAT = 128
D_EDGE = 16
NUM_GRAPHS = 16

NC, NS, L = 2, 16, 16
NW = NC * NS
NP = 10240
RPS = NP // NS
EPAD = 320512
EPW = EPAD // NW
CH = 32
NCHUNK = EPW // CH
EPADP = 327680
EPWP = EPADP // NW
CHP = 128
NCHUNKP = EPWP // CHP

_SC_MESH = plsc.VectorSubcoreMesh(
    core_axis_name="c", subcore_axis_name="s", num_cores=NC, num_subcores=NS)


def _mm2_body(x_ref, wa_ref, wb_ref, oa_ref, ob_ref):
    x = x_ref[...]
    dt = oa_ref.dtype
    oa_ref[...] = jnp.dot(x, wa_ref[...], preferred_element_type=jnp.float32).astype(dt)
    ob_ref[...] = jnp.dot(x, wb_ref[...], preferred_element_type=jnp.float32).astype(dt)


def _mm2(x, wa, wb, block_rows, out_dtype=jnp.float32):
    n, k = x.shape
    m = wa.shape[1]
    return pl.pallas_call(
        _mm2_body,
        grid=(n // block_rows,),
        in_specs=[
            pl.BlockSpec((block_rows, k), lambda i: (i, 0)),
            pl.BlockSpec((k, m), lambda i: (0, 0)),
            pl.BlockSpec((k, m), lambda i: (0, 0)),
        ],
        out_specs=[
            pl.BlockSpec((block_rows, m), lambda i: (i, 0)),
            pl.BlockSpec((block_rows, m), lambda i: (i, 0)),
        ],
        out_shape=[
            jax.ShapeDtypeStruct((n, m), out_dtype),
            jax.ShapeDtypeStruct((n, m), out_dtype),
        ],
    )(x, wa, wb)


def _prepass_body(dst_hbm, ea_hbm, deg_out, sa_out,
                  idxb, ea_v, ones_v, z32, sa_sh, deg_sh,
                  gsem0, gsem1, ssem0, ssem1, isem0, isem1):
    gsem = (gsem0, gsem1)
    ssem = (ssem0, ssem1)
    isem = (isem0, isem1)
    cid = lax.axis_index("c")
    sid = lax.axis_index("s")
    wid = sid * NC + cid
    ebase = wid * EPWP
    zv = jnp.zeros((L,), jnp.float32)

    def zea_body(i, _):
        ea_v[0, i, pl.ds(0, 16)] = zv
        return 0
    lax.fori_loop(0, CHP, zea_body, 0)
    for k in range(CHP // L):
        z32[pl.ds(k * L, L)] = zv
        ones_v[pl.ds(k * L, L)] = jnp.ones((L,), jnp.float32)
    for k in range(RPS // CHP):
        pltpu.sync_copy(ea_v.at[0], sa_sh.at[pl.ds(sid * RPS + k * CHP, CHP)])
        pltpu.sync_copy(z32, deg_sh.at[pl.ds(sid * RPS + k * CHP, CHP)])
    plsc.subcore_barrier()

    def idx_copy(c, sem, issue=True):
        base = pl.multiple_of(ebase + c * CHP, 8)
        s, d = dst_hbm.at[pl.ds(base, CHP)], idxb.at[c % 4]
        if issue:
            pltpu.async_copy(s, d, sem)
        else:
            pltpu.make_async_copy(s, d, sem).wait()

    def gathers(c, x, issue):
        base = pl.multiple_of(ebase + c * CHP, 8)
        s, d = ea_hbm.at[pl.ds(base, CHP)], ea_v.at[x]
        if issue:
            pltpu.async_copy(s, d, gsem[x])
        else:
            pltpu.make_async_copy(s, d, gsem[x]).wait()

    def scatters(c, x, issue):
        di = idxb.at[c % 4]
        ops = [(ea_v.at[x], sa_sh.at[di]), (ones_v, deg_sh.at[di])]
        for s, d in ops:
            if issue:
                pltpu.async_copy(s, d, ssem[x], add=True)
            else:
                pltpu.make_async_copy(s, d, ssem[x]).wait()

    idx_copy(0, isem[0])
    idx_copy(0, isem[0], issue=False)
    gathers(0, 0, issue=True)
    idx_copy(1, isem[1])

    NPAIR = (NCHUNKP - 1) // 2

    def pair_body(g, _):
        c = 2 * g
        for x in (0, 1):
            cc = c + x

            @pl.when(cc >= 2)
            def _():
                scatters(cc, x, issue=False)

            @pl.when(cc + 2 < NCHUNKP)
            def _():
                idx_copy(cc + 2, isem[x])

            @pl.when(cc + 1 < NCHUNKP)
            def _():
                idx_copy(cc + 1, isem[1 - x], issue=False)
                gathers(cc + 1, 1 - x, issue=True)

            gathers(cc, x, issue=False)
            scatters(cc, x, issue=True)
        return 0
    lax.fori_loop(0, NPAIR, pair_body, 0)

    for c in range(2 * NPAIR, NCHUNKP):
        x = c & 1
        scatters(c, x, issue=False)
        if c + 1 < NCHUNKP:
            idx_copy(c + 1, isem[1 - x], issue=False)
            gathers(c + 1, 1 - x, issue=True)
        gathers(c, x, issue=False)
        scatters(c, x, issue=True)
    scatters(NCHUNKP - 2, (NCHUNKP - 2) & 1, issue=False)
    scatters(NCHUNKP - 1, (NCHUNKP - 1) & 1, issue=False)

    plsc.subcore_barrier()
    pltpu.sync_copy(sa_sh.at[pl.ds(sid * RPS, RPS)],
                    sa_out.at[pl.ds(cid * NP + sid * RPS, RPS)])
    pltpu.sync_copy(deg_sh.at[pl.ds(sid * RPS, RPS)],
                    deg_out.at[pl.ds(cid * NP + sid * RPS, RPS)])


_sc_prepass = functools.partial(
    pl.kernel,
    out_type=(jax.ShapeDtypeStruct((NC * NP,), jnp.float32),
              jax.ShapeDtypeStruct((NC * NP, D_EDGE), jnp.float32)),
    mesh=_SC_MESH,
    scratch_types=[
        pltpu.VMEM((4, CHP), jnp.int32),
        pltpu.VMEM((2, CHP, D_EDGE), jnp.float32),
        pltpu.VMEM((CHP,), jnp.float32),
        pltpu.VMEM((CHP,), jnp.float32),
        pltpu.VMEM_SHARED((NP, D_EDGE), jnp.float32),
        pltpu.VMEM_SHARED((NP,), jnp.float32),
    ] + [pltpu.SemaphoreType.DMA] * 6,
)(_prepass_body)


def _edge_body(src_hbm, dst_hbm, xl_hbm, xr_hbm, e_hbm, att_hbm,
               u_out, den_out,
               idxb, xl_v, xr_v, e_v, u_v, exp_v, att_v,
               u_sh, den_sh,
               gsem0, gsem1, ssem0, ssem1, isem0, isem1):
    gsem = (gsem0, gsem1)
    ssem = (ssem0, ssem1)
    isem = (isem0, isem1)
    cid = lax.axis_index("c")
    sid = lax.axis_index("s")
    wid = sid * NC + cid
    ebase = wid * EPW
    zv = jnp.zeros((L,), jnp.float32)

    def zu_body(i, _):
        for j in range(D_FEAT // L):
            u_v[0, i, pl.ds(j * L, L)] = zv
        return 0
    lax.fori_loop(0, CH, zu_body, 0)
    for k in range(CH // L):
        exp_v[0, pl.ds(k * L, L)] = zv
    for k in range(RPS // CH):
        pltpu.sync_copy(u_v.at[0], u_sh.at[pl.ds(sid * RPS + k * CH, CH)])
        pltpu.sync_copy(exp_v.at[0], den_sh.at[pl.ds(sid * RPS + k * CH, CH)])
    rem = RPS - (RPS // CH) * CH
    if rem:
        pltpu.sync_copy(u_v.at[0, pl.ds(0, rem)],
                        u_sh.at[pl.ds(sid * RPS + RPS - rem, rem)])
        pltpu.sync_copy(exp_v.at[0, pl.ds(0, rem)],
                        den_sh.at[pl.ds(sid * RPS + RPS - rem, rem)])
    plsc.subcore_barrier()

    pltpu.sync_copy(att_hbm, att_v)

    def idx_copy(c, q, sem, issue=True):
        base = pl.multiple_of(ebase + c * CH, 8)
        ops = [
            (src_hbm.at[pl.ds(base, CH)], idxb.at[q, 0]),
            (dst_hbm.at[pl.ds(base, CH)], idxb.at[q, 1]),
        ]
        for s, d in ops:
            if issue:
                pltpu.async_copy(s, d, sem)
            else:
                pltpu.make_async_copy(s, d, sem).wait()

    def gathers(c, x, issue):
        q = c % 4
        base = pl.multiple_of(ebase + c * CH, 8)
        ops = [
            (xl_hbm.at[idxb.at[q, 0]], xl_v.at[x]),
            (xr_hbm.at[idxb.at[q, 1]], xr_v.at[x]),
            (e_hbm.at[pl.ds(base, CH)], e_v.at[x]),
        ]
        for s, d in ops:
            if issue:
                pltpu.async_copy(s, d, gsem[x])
            else:
                pltpu.make_async_copy(s, d, gsem[x]).wait()

    def scatters(c, x, issue):
        di = idxb.at[c % 4, 1]
        ops = [
            (u_v.at[x], u_sh.at[di]),
            (exp_v.at[x], den_sh.at[di]),
        ]
        for s, d in ops:
            if issue:
                pltpu.async_copy(s, d, ssem[x], add=True)
            else:
                pltpu.make_async_copy(s, d, ssem[x]).wait()

    lane = lax.iota(jnp.int32, L)
    _ib = "promise_in_bounds"

    def _allsum(v):
        for k in (8, 4, 2, 1):
            v = v + v.at[lane ^ k].get(mode=_ib)
        return v

    def compute(x):
        r = tuple(att_v[pl.ds(j * L, L)] for j in range(D_FEAT // L))

        def group_body(g, _):
            exg = jnp.zeros((L,), jnp.float32)
            for i in range(L):
                e = g * L + i
                acc = jnp.zeros((L,), jnp.float32)
                xls = []
                for j in range(D_FEAT // L):
                    xlj = xl_v[x, e, pl.ds(j * L, L)]
                    m = (xlj + xr_v[x, e, pl.ds(j * L, L)]
                         + e_v[x, e, pl.ds(j * L, L)])
                    m = jnp.maximum(m, m * 0.2)
                    acc = acc + m * r[j]
                    xls.append(xlj)
                exs = jnp.exp(_allsum(acc))
                for j in range(D_FEAT // L):
                    u_v[x, e, pl.ds(j * L, L)] = xls[j] * exs
                exg = jnp.where(lane == i, exs, exg)
            exp_v[x, pl.ds(g * L, L)] = exg
            return 0
        lax.fori_loop(0, CH // L, group_body, 0)

    idx_copy(0, 0, isem[0])
    idx_copy(0, 0, isem[0], issue=False)
    gathers(0, 0, issue=True)
    idx_copy(1, 1, isem[1])

    def pair_body(g, _):
        c = 2 * g
        for x in (0, 1):
            cc = c + x

            @pl.when(cc >= 2)
            def _():
                scatters(cc, x, issue=False)

            @pl.when(cc + 2 < NCHUNK)
            def _():
                idx_copy(cc + 2, (cc + 2) % 4, isem[x])

            @pl.when(cc + 1 < NCHUNK)
            def _():
                idx_copy(cc + 1, (cc + 1) % 4, isem[1 - x], issue=False)
                gathers(cc + 1, 1 - x, issue=True)

            gathers(cc, x, issue=False)
            compute(x)
            scatters(cc, x, issue=True)
        return 0
    lax.fori_loop(0, (NCHUNK - 1) // 2, pair_body, 0)

    ct = NCHUNK - 1
    scatters(ct, 0, issue=False)
    gathers(ct, 0, issue=False)
    compute(0)
    scatters(ct, 0, issue=True)
    scatters(ct - 1, 1, issue=False)
    scatters(ct, 0, issue=False)

    plsc.subcore_barrier()
    pltpu.sync_copy(u_sh.at[pl.ds(sid * RPS, RPS)],
                    u_out.at[pl.ds(cid * NP + sid * RPS, RPS)])
    pltpu.sync_copy(den_sh.at[pl.ds(sid * RPS, RPS)],
                    den_out.at[pl.ds(cid * NP + sid * RPS, RPS)])


_sc_edge_pass = functools.partial(
    pl.kernel,
    out_type=(jax.ShapeDtypeStruct((NC * NP, D_FEAT), jnp.float32),
              jax.ShapeDtypeStruct((NC * NP,), jnp.float32)),
    mesh=_SC_MESH,
    scratch_types=[
        pltpu.VMEM((4, 2, CH), jnp.int32),
        pltpu.VMEM((2, CH, D_FEAT), jnp.float32),
        pltpu.VMEM((2, CH, D_FEAT), jnp.float32),
        pltpu.VMEM((2, CH, D_FEAT), jnp.float32),
        pltpu.VMEM((2, CH, D_FEAT), jnp.float32),
        pltpu.VMEM((2, CH), jnp.float32),
        pltpu.VMEM((D_FEAT,), jnp.float32),
        pltpu.VMEM_SHARED((NP, D_FEAT), jnp.float32),
        pltpu.VMEM_SHARED((NP,), jnp.float32),
    ] + [pltpu.SemaphoreType.DMA] * 6,
)(_edge_body)


def _interlude_body(u_ref, den_ref, xl_ref, xr_ref, sa_ref, deg_ref,
                    we_ref, att_ref, b_ref, o_ref):
    xl = xl_ref[...]
    xr = xr_ref[...]
    u = u_ref[0] + u_ref[1]
    deg = jnp.maximum(deg_ref[0] + deg_ref[1], 1.0)
    loop_attr = (sa_ref[0] + sa_ref[1]) / deg[:, None]
    loop128 = jnp.dot(loop_attr, we_ref[...], preferred_element_type=jnp.float32)
    m = xl + xr + loop128
    m = jnp.maximum(m, m * 0.2)
    logit = jnp.sum(m * att_ref[...], axis=1)
    es = jnp.exp(logit)
    dt = den_ref[0] + den_ref[1] + es + 1e-16
    x = (u + es[:, None] * xl) / dt[:, None] + b_ref[...]
    o_ref[...] = jnp.maximum(x, 0.0)


def _interlude(U, den, XL, XR, sa, deg, We, att, b, block_rows=2048):
    g = NP // block_rows
    return pl.pallas_call(
        _interlude_body,
        grid=(g,),
        in_specs=[
            pl.BlockSpec((NC, block_rows, D_FEAT), lambda i: (0, i, 0)),
            pl.BlockSpec((NC, block_rows), lambda i: (0, i)),
            pl.BlockSpec((block_rows, D_FEAT), lambda i: (i, 0)),
            pl.BlockSpec((block_rows, D_FEAT), lambda i: (i, 0)),
            pl.BlockSpec((NC, block_rows, D_EDGE), lambda i: (0, i, 0)),
            pl.BlockSpec((NC, block_rows), lambda i: (0, i)),
            pl.BlockSpec((D_EDGE, D_FEAT), lambda i: (0, 0)),
            pl.BlockSpec((1, D_FEAT), lambda i: (0, 0)),
            pl.BlockSpec((1, D_FEAT), lambda i: (0, 0)),
        ],
        out_specs=pl.BlockSpec((block_rows, D_FEAT), lambda i: (i, 0)),
        out_shape=jax.ShapeDtypeStruct((NP, D_FEAT), jnp.float32),
    )(U.reshape(NC, NP, D_FEAT), den.reshape(NC, NP), XL, XR,
      sa.reshape(NC, NP, D_EDGE), deg.reshape(NC, NP), We,
      att.reshape(1, D_FEAT), b.reshape(1, D_FEAT))


def _pool_body(x_ref, b_ref, o_ref):
    x = x_ref[...]
    b = b_ref[...]
    gids = lax.broadcasted_iota(jnp.int32, (NUM_GRAPHS, N_NODES), 0)
    onehot = (b == gids).astype(jnp.float32)
    sums = jnp.dot(onehot, x, preferred_element_type=jnp.float32)
    counts = jnp.sum(onehot, axis=1)
    o_ref[...] = sums / jnp.maximum(counts, 1.0)[:, None]


def _mean_pool(x, batch):
    return pl.pallas_call(
        _pool_body,
        in_specs=[
            pl.BlockSpec((N_NODES, D_FEAT), lambda: (0, 0)),
            pl.BlockSpec((1, N_NODES), lambda: (0, 0)),
        ],
        out_specs=pl.BlockSpec((NUM_GRAPHS, D_FEAT), lambda: (0, 0)),
        out_shape=jax.ShapeDtypeStruct((NUM_GRAPHS, D_FEAT), jnp.float32),
    )(x, batch.reshape(1, N_NODES))


def kernel(node_features, edge_index, batch, edge_attr, Wl1, Wr1, We1, att1, b1, Wl2, Wr2, We2, att2, b2):
    x0 = jnp.pad(node_features, ((0, NP - N_NODES), (0, 0)))
    npad = EPAD - N_EDGES
    src = jnp.pad(edge_index[0], (0, npad))
    dst = jnp.pad(edge_index[1], (0, npad), constant_values=NP - 1)
    ea = jnp.pad(edge_attr, ((0, npad), (0, 0)))

    dst_p = jnp.pad(edge_index[1], (0, EPADP - N_EDGES), constant_values=NP - 1)
    ea_p = jnp.pad(edge_attr, ((0, EPADP - N_EDGES), (0, 0)))
    deg_p, sa_p = _sc_prepass(dst_p, ea_p)
    E1, E2 = _mm2(ea, We1, We2, 1024)

    XL1, XR1 = _mm2(x0, Wl1, Wr1, 2048)
    U1, den1 = _sc_edge_pass(src, dst, XL1, XR1, E1, att1)
    x1 = _interlude(U1, den1, XL1, XR1, sa_p, deg_p, We1, att1, b1)

    XL2, XR2 = _mm2(x1, Wl2, Wr2, 2048)
    U2, den2 = _sc_edge_pass(src, dst, XL2, XR2, E2, att2)
    x2 = _interlude(U2, den2, XL2, XR2, sa_p, deg_p, We2, att2, b2)

    return _mean_pool(x2[:N_NODES], batch)


# speedup vs baseline: 1.0223x; 1.0223x over previous
import functools

import jax
import jax.numpy as jnp
from jax import lax
from jax.experimental import pallas as pl
from jax.experimental.pallas import tpu as pltpu
from jax.experimental.pallas import tpu_sc as plsc

N_NODES = 10000
N_EDGES = 320000
D_FEAT = 128
D_EDGE = 16
NUM_GRAPHS = 16

NC, NS, L = 2, 16, 16
NW = NC * NS
NP = 10240
RPS = NP // NS
EPAD = 320512
EPW = EPAD // NW
CH = 32
NCHUNK = EPW // CH
EPWP = EPW
CHP = 32
NCHUNKP = EPWP // CHP

_SC_MESH = plsc.VectorSubcoreMesh(
    core_axis_name="c", subcore_axis_name="s", num_cores=NC, num_subcores=NS)


def _mm2_body(x_ref, wa_ref, wb_ref, oa_ref, ob_ref):
    x = x_ref[...]
    dt = oa_ref.dtype
    oa_ref[...] = jnp.dot(x, wa_ref[...], preferred_element_type=jnp.float32).astype(dt)
    ob_ref[...] = jnp.dot(x, wb_ref[...], preferred_element_type=jnp.float32).astype(dt)


def _mm2(x, wa, wb, block_rows, out_dtype=jnp.float32):
    n, k = x.shape
    m = wa.shape[1]
    return pl.pallas_call(
        _mm2_body,
        grid=(n // block_rows,),
        in_specs=[
            pl.BlockSpec((block_rows, k), lambda i: (i, 0)),
            pl.BlockSpec((k, m), lambda i: (0, 0)),
            pl.BlockSpec((k, m), lambda i: (0, 0)),
        ],
        out_specs=[
            pl.BlockSpec((block_rows, m), lambda i: (i, 0)),
            pl.BlockSpec((block_rows, m), lambda i: (i, 0)),
        ],
        out_shape=[
            jax.ShapeDtypeStruct((n, m), out_dtype),
            jax.ShapeDtypeStruct((n, m), out_dtype),
        ],
    )(x, wa, wb)


def _prepass_body(dst_hbm, ea_hbm, deg_out, sa_out,
                  idxb, ea_v, ones_v, z32, sa_sh, deg_sh,
                  gsem0, gsem1, ssem0, ssem1, isem0, isem1):
    gsem = (gsem0, gsem1)
    ssem = (ssem0, ssem1)
    isem = (isem0, isem1)
    cid = lax.axis_index("c")
    sid = lax.axis_index("s")
    wid = sid * NC + cid
    ebase = wid * EPWP
    zv = jnp.zeros((L,), jnp.float32)

    def zea_body(i, _):
        ea_v[0, i, pl.ds(0, 16)] = zv
        return 0
    lax.fori_loop(0, CHP, zea_body, 0)
    for k in range(CHP // L):
        z32[pl.ds(k * L, L)] = zv
        ones_v[pl.ds(k * L, L)] = jnp.ones((L,), jnp.float32)
    for k in range(RPS // CHP):
        pltpu.sync_copy(ea_v.at[0], sa_sh.at[pl.ds(sid * RPS + k * CHP, CHP)])
        pltpu.sync_copy(z32, deg_sh.at[pl.ds(sid * RPS + k * CHP, CHP)])
    plsc.subcore_barrier()

    def idx_copy(c, sem, issue=True):
        base = pl.multiple_of(ebase + c * CHP, 8)
        s, d = dst_hbm.at[pl.ds(base, CHP)], idxb.at[c % 4]
        if issue:
            pltpu.async_copy(s, d, sem)
        else:
            pltpu.make_async_copy(s, d, sem).wait()

    def gathers(c, x, issue):
        base = pl.multiple_of(ebase + c * CHP, 8)
        s, d = ea_hbm.at[pl.ds(base, CHP)], ea_v.at[x]
        if issue:
            pltpu.async_copy(s, d, gsem[x])
        else:
            pltpu.make_async_copy(s, d, gsem[x]).wait()

    def scatters(c, x, issue):
        di = idxb.at[c % 4]
        ops = [(ea_v.at[x], sa_sh.at[di]), (ones_v, deg_sh.at[di])]
        for s, d in ops:
            if issue:
                pltpu.async_copy(s, d, ssem[x], add=True)
            else:
                pltpu.make_async_copy(s, d, ssem[x]).wait()

    idx_copy(0, isem[0])
    idx_copy(0, isem[0], issue=False)
    gathers(0, 0, issue=True)
    idx_copy(1, isem[1])

    NPAIR = (NCHUNKP - 1) // 2

    def pair_body(g, _):
        c = 2 * g
        for x in (0, 1):
            cc = c + x

            @pl.when(cc >= 2)
            def _():
                scatters(cc, x, issue=False)

            @pl.when(cc + 2 < NCHUNKP)
            def _():
                idx_copy(cc + 2, isem[x])

            @pl.when(cc + 1 < NCHUNKP)
            def _():
                idx_copy(cc + 1, isem[1 - x], issue=False)
                gathers(cc + 1, 1 - x, issue=True)

            gathers(cc, x, issue=False)
            scatters(cc, x, issue=True)
        return 0
    lax.fori_loop(0, NPAIR, pair_body, 0)

    for c in range(2 * NPAIR, NCHUNKP):
        x = c & 1
        scatters(c, x, issue=False)
        if c + 1 < NCHUNKP:
            idx_copy(c + 1, isem[1 - x], issue=False)
            gathers(c + 1, 1 - x, issue=True)
        gathers(c, x, issue=False)
        scatters(c, x, issue=True)
    scatters(NCHUNKP - 2, (NCHUNKP - 2) & 1, issue=False)
    scatters(NCHUNKP - 1, (NCHUNKP - 1) & 1, issue=False)

    plsc.subcore_barrier()
    pltpu.sync_copy(sa_sh.at[pl.ds(sid * RPS, RPS)],
                    sa_out.at[pl.ds(cid * NP + sid * RPS, RPS)])
    pltpu.sync_copy(deg_sh.at[pl.ds(sid * RPS, RPS)],
                    deg_out.at[pl.ds(cid * NP + sid * RPS, RPS)])


_sc_prepass = functools.partial(
    pl.kernel,
    out_type=(jax.ShapeDtypeStruct((NC * NP,), jnp.float32),
              jax.ShapeDtypeStruct((NC * NP, D_EDGE), jnp.float32)),
    mesh=_SC_MESH,
    scratch_types=[
        pltpu.VMEM((4, CHP), jnp.int32),
        pltpu.VMEM((2, CHP, D_EDGE), jnp.float32),
        pltpu.VMEM((CHP,), jnp.float32),
        pltpu.VMEM((CHP,), jnp.float32),
        pltpu.VMEM_SHARED((NP, D_EDGE), jnp.float32),
        pltpu.VMEM_SHARED((NP,), jnp.float32),
    ] + [pltpu.SemaphoreType.DMA] * 6,
)(_prepass_body)


def _edge_body(src_hbm, dst_hbm, xl_hbm, xr_hbm, e_hbm, att_hbm,
               u_out, den_out,
               idxb, xl_v, xr_v, e_v, u_v, exp_v, att_v,
               u_sh, den_sh,
               gsem0, gsem1, ssem0, ssem1, isem0, isem1):
    gsem = (gsem0, gsem1)
    ssem = (ssem0, ssem1)
    isem = (isem0, isem1)
    cid = lax.axis_index("c")
    sid = lax.axis_index("s")
    wid = sid * NC + cid
    ebase = wid * EPW
    zv = jnp.zeros((L,), jnp.float32)

    def zu_body(i, _):
        for j in range(D_FEAT // L):
            u_v[0, i, pl.ds(j * L, L)] = zv
        return 0
    lax.fori_loop(0, CH, zu_body, 0)
    for k in range(CH // L):
        exp_v[0, pl.ds(k * L, L)] = zv
    for k in range(RPS // CH):
        pltpu.sync_copy(u_v.at[0], u_sh.at[pl.ds(sid * RPS + k * CH, CH)])
        pltpu.sync_copy(exp_v.at[0], den_sh.at[pl.ds(sid * RPS + k * CH, CH)])
    rem = RPS - (RPS // CH) * CH
    if rem:
        pltpu.sync_copy(u_v.at[0, pl.ds(0, rem)],
                        u_sh.at[pl.ds(sid * RPS + RPS - rem, rem)])
        pltpu.sync_copy(exp_v.at[0, pl.ds(0, rem)],
                        den_sh.at[pl.ds(sid * RPS + RPS - rem, rem)])
    plsc.subcore_barrier()

    pltpu.sync_copy(att_hbm, att_v)

    def idx_copy(c, q, sem, issue=True):
        base = pl.multiple_of(ebase + c * CH, 8)
        ops = [
            (src_hbm.at[pl.ds(base, CH)], idxb.at[q, 0]),
            (dst_hbm.at[pl.ds(base, CH)], idxb.at[q, 1]),
        ]
        for s, d in ops:
            if issue:
                pltpu.async_copy(s, d, sem)
            else:
                pltpu.make_async_copy(s, d, sem).wait()

    def gathers(c, x, issue):
        q = c % 4
        base = pl.multiple_of(ebase + c * CH, 8)
        ops = [
            (xl_hbm.at[idxb.at[q, 0]], xl_v.at[x]),
            (xr_hbm.at[idxb.at[q, 1]], xr_v.at[x]),
            (e_hbm.at[pl.ds(base, CH)], e_v.at[x]),
        ]
        for s, d in ops:
            if issue:
                pltpu.async_copy(s, d, gsem[x])
            else:
                pltpu.make_async_copy(s, d, gsem[x]).wait()

    def scatters(c, x, issue):
        di = idxb.at[c % 4, 1]
        ops = [
            (u_v.at[x], u_sh.at[di]),
            (exp_v.at[x], den_sh.at[di]),
        ]
        for s, d in ops:
            if issue:
                pltpu.async_copy(s, d, ssem[x], add=True)
            else:
                pltpu.make_async_copy(s, d, ssem[x]).wait()

    lane = lax.iota(jnp.int32, L)
    _ib = "promise_in_bounds"

    def _allsum(v):
        for k in (8, 4, 2, 1):
            v = v + v.at[lane ^ k].get(mode=_ib)
        return v

    def compute(x):
        r = tuple(att_v[pl.ds(j * L, L)] for j in range(D_FEAT // L))

        def group_body(g, _):
            exg = jnp.zeros((L,), jnp.float32)
            for i in range(L):
                e = g * L + i
                acc = jnp.zeros((L,), jnp.float32)
                xls = []
                for j in range(D_FEAT // L):
                    xlj = xl_v[x, e, pl.ds(j * L, L)]
                    m = (xlj + xr_v[x, e, pl.ds(j * L, L)]
                         + e_v[x, e, pl.ds(j * L, L)])
                    m = jnp.maximum(m, m * 0.2)
                    acc = acc + m * r[j]
                    xls.append(xlj)
                exs = jnp.exp(_allsum(acc))
                for j in range(D_FEAT // L):
                    u_v[x, e, pl.ds(j * L, L)] = xls[j] * exs
                exg = jnp.where(lane == i, exs, exg)
            exp_v[x, pl.ds(g * L, L)] = exg
            return 0
        lax.fori_loop(0, CH // L, group_body, 0)

    idx_copy(0, 0, isem[0])
    idx_copy(0, 0, isem[0], issue=False)
    gathers(0, 0, issue=True)
    idx_copy(1, 1, isem[1])

    def pair_body(g, _):
        c = 2 * g
        for x in (0, 1):
            cc = c + x

            @pl.when(cc >= 2)
            def _():
                scatters(cc, x, issue=False)

            @pl.when(cc + 2 < NCHUNK)
            def _():
                idx_copy(cc + 2, (cc + 2) % 4, isem[x])

            @pl.when(cc + 1 < NCHUNK)
            def _():
                idx_copy(cc + 1, (cc + 1) % 4, isem[1 - x], issue=False)
                gathers(cc + 1, 1 - x, issue=True)

            gathers(cc, x, issue=False)
            compute(x)
            scatters(cc, x, issue=True)
        return 0
    lax.fori_loop(0, (NCHUNK - 1) // 2, pair_body, 0)

    ct = NCHUNK - 1
    scatters(ct, 0, issue=False)
    gathers(ct, 0, issue=False)
    compute(0)
    scatters(ct, 0, issue=True)
    scatters(ct - 1, 1, issue=False)
    scatters(ct, 0, issue=False)

    plsc.subcore_barrier()
    pltpu.sync_copy(u_sh.at[pl.ds(sid * RPS, RPS)],
                    u_out.at[pl.ds(cid * NP + sid * RPS, RPS)])
    pltpu.sync_copy(den_sh.at[pl.ds(sid * RPS, RPS)],
                    den_out.at[pl.ds(cid * NP + sid * RPS, RPS)])


_sc_edge_pass = functools.partial(
    pl.kernel,
    out_type=(jax.ShapeDtypeStruct((NC * NP, D_FEAT), jnp.float32),
              jax.ShapeDtypeStruct((NC * NP,), jnp.float32)),
    mesh=_SC_MESH,
    scratch_types=[
        pltpu.VMEM((4, 2, CH), jnp.int32),
        pltpu.VMEM((2, CH, D_FEAT), jnp.float32),
        pltpu.VMEM((2, CH, D_FEAT), jnp.float32),
        pltpu.VMEM((2, CH, D_FEAT), jnp.float32),
        pltpu.VMEM((2, CH, D_FEAT), jnp.float32),
        pltpu.VMEM((2, CH), jnp.float32),
        pltpu.VMEM((D_FEAT,), jnp.float32),
        pltpu.VMEM_SHARED((NP, D_FEAT), jnp.float32),
        pltpu.VMEM_SHARED((NP,), jnp.float32),
    ] + [pltpu.SemaphoreType.DMA] * 6,
)(_edge_body)


def _node_x(u_ref, den_ref, xl_ref, xr_ref, sa_ref, deg_ref,
            we_ref, att_ref, b_ref):
    xl = xl_ref[...]
    xr = xr_ref[...]
    u = u_ref[0] + u_ref[1]
    deg = jnp.maximum(deg_ref[0] + deg_ref[1], 1.0)
    loop_attr = (sa_ref[0] + sa_ref[1]) / deg[:, None]
    loop128 = jnp.dot(loop_attr, we_ref[...], preferred_element_type=jnp.float32)
    m = xl + xr + loop128
    m = jnp.maximum(m, m * 0.2)
    logit = jnp.sum(m * att_ref[...], axis=1)
    es = jnp.exp(logit)
    dt = den_ref[0] + den_ref[1] + es + 1e-16
    x = (u + es[:, None] * xl) / dt[:, None] + b_ref[...]
    return jnp.maximum(x, 0.0)


_ILU_SPECS = [
    pl.BlockSpec((NC, 2048, D_FEAT), lambda i: (0, i, 0)),
    pl.BlockSpec((NC, 2048), lambda i: (0, i)),
    pl.BlockSpec((2048, D_FEAT), lambda i: (i, 0)),
    pl.BlockSpec((2048, D_FEAT), lambda i: (i, 0)),
    pl.BlockSpec((NC, 2048, D_EDGE), lambda i: (0, i, 0)),
    pl.BlockSpec((NC, 2048), lambda i: (0, i)),
    pl.BlockSpec((D_EDGE, D_FEAT), lambda i: (0, 0)),
    pl.BlockSpec((1, D_FEAT), lambda i: (0, 0)),
    pl.BlockSpec((1, D_FEAT), lambda i: (0, 0)),
]


def _ilu_args(U, den, XL, XR, sa, deg, We, att, b):
    return (U.reshape(NC, NP, D_FEAT), den.reshape(NC, NP), XL, XR,
            sa.reshape(NC, NP, D_EDGE), deg.reshape(NC, NP), We,
            att.reshape(1, D_FEAT), b.reshape(1, D_FEAT))


def _interlude_mm_body(u_ref, den_ref, xl_ref, xr_ref, sa_ref, deg_ref,
                       we_ref, att_ref, b_ref, wl_ref, wr_ref,
                       xl2_ref, xr2_ref):
    x = _node_x(u_ref, den_ref, xl_ref, xr_ref, sa_ref, deg_ref,
                we_ref, att_ref, b_ref)
    xl2_ref[...] = jnp.dot(x, wl_ref[...], preferred_element_type=jnp.float32)
    xr2_ref[...] = jnp.dot(x, wr_ref[...], preferred_element_type=jnp.float32)


def _interlude_mm(U, den, XL, XR, sa, deg, We, att, b, Wl2, Wr2):
    return pl.pallas_call(
        _interlude_mm_body,
        grid=(NP // 2048,),
        in_specs=_ILU_SPECS + [
            pl.BlockSpec((D_FEAT, D_FEAT), lambda i: (0, 0)),
            pl.BlockSpec((D_FEAT, D_FEAT), lambda i: (0, 0)),
        ],
        out_specs=[
            pl.BlockSpec((2048, D_FEAT), lambda i: (i, 0)),
            pl.BlockSpec((2048, D_FEAT), lambda i: (i, 0)),
        ],
        out_shape=[
            jax.ShapeDtypeStruct((NP, D_FEAT), jnp.float32),
            jax.ShapeDtypeStruct((NP, D_FEAT), jnp.float32),
        ],
    )(*_ilu_args(U, den, XL, XR, sa, deg, We, att, b), Wl2, Wr2)


def _interlude_pool_body(u_ref, den_ref, xl_ref, xr_ref, sa_ref, deg_ref,
                         we_ref, att_ref, b_ref, batch_ref, o_ref,
                         sum_acc, cnt_acc):
    i = pl.program_id(0)
    x = _node_x(u_ref, den_ref, xl_ref, xr_ref, sa_ref, deg_ref,
                we_ref, att_ref, b_ref)
    bt = batch_ref[...]
    gids = lax.broadcasted_iota(jnp.int32, (NUM_GRAPHS, 2048), 0)
    onehot = (bt == gids).astype(jnp.float32)
    ps = jnp.dot(onehot, x, preferred_element_type=jnp.float32)
    pc = jnp.sum(onehot, axis=1)

    @pl.when(i == 0)
    def _():
        sum_acc[...] = jnp.zeros_like(sum_acc)
        cnt_acc[...] = jnp.zeros_like(cnt_acc)
    sum_acc[...] += ps
    cnt_acc[...] += pc[:, None]

    @pl.when(i == NP // 2048 - 1)
    def _():
        o_ref[...] = sum_acc[...] / jnp.maximum(cnt_acc[...], 1.0)


def _interlude_pool(U, den, XL, XR, sa, deg, We, att, b, batch):
    return pl.pallas_call(
        _interlude_pool_body,
        grid=(NP // 2048,),
        in_specs=_ILU_SPECS + [pl.BlockSpec((1, 2048), lambda i: (0, i))],
        out_specs=pl.BlockSpec((NUM_GRAPHS, D_FEAT), lambda i: (0, 0)),
        out_shape=jax.ShapeDtypeStruct((NUM_GRAPHS, D_FEAT), jnp.float32),
        scratch_shapes=[
            pltpu.VMEM((NUM_GRAPHS, D_FEAT), jnp.float32),
            pltpu.VMEM((NUM_GRAPHS, 1), jnp.float32),
        ],
    )(*_ilu_args(U, den, XL, XR, sa, deg, We, att, b),
      batch.reshape(1, NP))


def kernel(node_features, edge_index, batch, edge_attr, Wl1, Wr1, We1, att1, b1, Wl2, Wr2, We2, att2, b2):
    x0 = jnp.pad(node_features, ((0, NP - N_NODES), (0, 0)))
    npad = EPAD - N_EDGES
    src = jnp.pad(edge_index[0], (0, npad))
    dst = jnp.pad(edge_index[1], (0, npad), constant_values=NP - 1)
    ea = jnp.pad(edge_attr, ((0, npad), (0, 0)))
    batch_p = jnp.pad(batch, (0, NP - N_NODES), constant_values=NUM_GRAPHS)

    deg_p, sa_p = _sc_prepass(dst, ea)
    E1, E2 = _mm2(ea, We1, We2, 1024)

    XL1, XR1 = _mm2(x0, Wl1, Wr1, 2048)
    U1, den1 = _sc_edge_pass(src, dst, XL1, XR1, E1, att1)
    XL2, XR2 = _interlude_mm(U1, den1, XL1, XR1, sa_p, deg_p, We1, att1, b1,
                             Wl2, Wr2)

    U2, den2 = _sc_edge_pass(src, dst, XL2, XR2, E2, att2)
    return _interlude_pool(U2, den2, XL2, XR2, sa_p, deg_p, We2, att2, b2,
                           batch_p)


# speedup vs baseline: 1.1196x; 1.0952x over previous
import functools

import jax
import jax.numpy as jnp
from jax import lax
from jax.experimental import pallas as pl
from jax.experimental.pallas import tpu as pltpu
from jax.experimental.pallas import tpu_sc as plsc

N_NODES = 10000
N_EDGES = 320000
D_FEAT = 128
D_EDGE = 16
NUM_GRAPHS = 16

NC, NS, L = 2, 16, 16
NW = NC * NS
NP = 10240
RPS = NP // NS
EPAD = 320512
EPW = EPAD // NW
CH = 32
NCHUNK = EPW // CH
EPWP = EPW
CHP = 32
NCHUNKP = EPWP // CHP

_SC_MESH = plsc.VectorSubcoreMesh(
    core_axis_name="c", subcore_axis_name="s", num_cores=NC, num_subcores=NS)


def _mm2_body(x_ref, wa_ref, wb_ref, oa_ref, ob_ref):
    x = x_ref[...]
    dt = oa_ref.dtype
    oa_ref[...] = jnp.dot(x, wa_ref[...], preferred_element_type=jnp.float32).astype(dt)
    ob_ref[...] = jnp.dot(x, wb_ref[...], preferred_element_type=jnp.float32).astype(dt)


def _mm2(x, wa, wb, block_rows, out_dtype=jnp.float32):
    n, k = x.shape
    m = wa.shape[1]
    return pl.pallas_call(
        _mm2_body,
        grid=(n // block_rows,),
        in_specs=[
            pl.BlockSpec((block_rows, k), lambda i: (i, 0)),
            pl.BlockSpec((k, m), lambda i: (0, 0)),
            pl.BlockSpec((k, m), lambda i: (0, 0)),
        ],
        out_specs=[
            pl.BlockSpec((block_rows, m), lambda i: (i, 0)),
            pl.BlockSpec((block_rows, m), lambda i: (i, 0)),
        ],
        out_shape=[
            jax.ShapeDtypeStruct((n, m), out_dtype),
            jax.ShapeDtypeStruct((n, m), out_dtype),
        ],
    )(x, wa, wb)


def _prepass_body(dst_hbm, ea_hbm, deg_out, sa_out,
                  idxb, ea_v, ones_v, z32, sa_sh, deg_sh,
                  gsem0, gsem1, ssem0, ssem1, isem0, isem1):
    gsem = (gsem0, gsem1)
    ssem = (ssem0, ssem1)
    isem = (isem0, isem1)
    cid = lax.axis_index("c")
    sid = lax.axis_index("s")
    wid = sid * NC + cid
    ebase = wid * EPWP
    zv = jnp.zeros((L,), jnp.float32)

    def zea_body(i, _):
        ea_v[0, i, pl.ds(0, 16)] = zv
        return 0
    lax.fori_loop(0, CHP, zea_body, 0)
    for k in range(CHP // L):
        z32[pl.ds(k * L, L)] = zv
        ones_v[pl.ds(k * L, L)] = jnp.ones((L,), jnp.float32)
    for k in range(RPS // CHP):
        pltpu.sync_copy(ea_v.at[0], sa_sh.at[pl.ds(sid * RPS + k * CHP, CHP)])
        pltpu.sync_copy(z32, deg_sh.at[pl.ds(sid * RPS + k * CHP, CHP)])
    plsc.subcore_barrier()

    def idx_copy(c, sem, issue=True):
        base = pl.multiple_of(ebase + c * CHP, 8)
        s, d = dst_hbm.at[pl.ds(base, CHP)], idxb.at[c % 4]
        if issue:
            pltpu.async_copy(s, d, sem)
        else:
            pltpu.make_async_copy(s, d, sem).wait()

    def gathers(c, x, issue):
        base = pl.multiple_of(ebase + c * CHP, 8)
        s, d = ea_hbm.at[pl.ds(base, CHP)], ea_v.at[x]
        if issue:
            pltpu.async_copy(s, d, gsem[x])
        else:
            pltpu.make_async_copy(s, d, gsem[x]).wait()

    def scatters(c, x, issue):
        di = idxb.at[c % 4]
        ops = [(ea_v.at[x], sa_sh.at[di]), (ones_v, deg_sh.at[di])]
        for s, d in ops:
            if issue:
                pltpu.async_copy(s, d, ssem[x], add=True)
            else:
                pltpu.make_async_copy(s, d, ssem[x]).wait()

    idx_copy(0, isem[0])
    idx_copy(0, isem[0], issue=False)
    gathers(0, 0, issue=True)
    idx_copy(1, isem[1])

    NPAIR = (NCHUNKP - 1) // 2

    def pair_body(g, _):
        c = 2 * g
        for x in (0, 1):
            cc = c + x

            @pl.when(cc >= 2)
            def _():
                scatters(cc, x, issue=False)

            @pl.when(cc + 2 < NCHUNKP)
            def _():
                idx_copy(cc + 2, isem[x])

            @pl.when(cc + 1 < NCHUNKP)
            def _():
                idx_copy(cc + 1, isem[1 - x], issue=False)
                gathers(cc + 1, 1 - x, issue=True)

            gathers(cc, x, issue=False)
            scatters(cc, x, issue=True)
        return 0
    lax.fori_loop(0, NPAIR, pair_body, 0)

    for c in range(2 * NPAIR, NCHUNKP):
        x = c & 1
        scatters(c, x, issue=False)
        if c + 1 < NCHUNKP:
            idx_copy(c + 1, isem[1 - x], issue=False)
            gathers(c + 1, 1 - x, issue=True)
        gathers(c, x, issue=False)
        scatters(c, x, issue=True)
    scatters(NCHUNKP - 2, (NCHUNKP - 2) & 1, issue=False)
    scatters(NCHUNKP - 1, (NCHUNKP - 1) & 1, issue=False)

    plsc.subcore_barrier()
    pltpu.sync_copy(sa_sh.at[pl.ds(sid * RPS, RPS)],
                    sa_out.at[pl.ds(cid * NP + sid * RPS, RPS)])
    pltpu.sync_copy(deg_sh.at[pl.ds(sid * RPS, RPS)],
                    deg_out.at[pl.ds(cid * NP + sid * RPS, RPS)])


_sc_prepass = functools.partial(
    pl.kernel,
    out_type=(jax.ShapeDtypeStruct((NC * NP,), jnp.float32),
              jax.ShapeDtypeStruct((NC * NP, D_EDGE), jnp.float32)),
    mesh=_SC_MESH,
    scratch_types=[
        pltpu.VMEM((4, CHP), jnp.int32),
        pltpu.VMEM((2, CHP, D_EDGE), jnp.float32),
        pltpu.VMEM((CHP,), jnp.float32),
        pltpu.VMEM((CHP,), jnp.float32),
        pltpu.VMEM_SHARED((NP, D_EDGE), jnp.float32),
        pltpu.VMEM_SHARED((NP,), jnp.float32),
    ] + [pltpu.SemaphoreType.DMA] * 6,
)(_prepass_body)


def _edge_body(sd_hbm, xl_hbm, xr_hbm, e_hbm, att_hbm,
               u_out, den_out,
               idxb, xl_v, xr_v, e_v, u_v, exp_v, att_v,
               u_sh, den_sh,
               gsem0, gsem1, ssem0, ssem1, isem0, isem1):
    gsem = (gsem0, gsem1)
    ssem = (ssem0, ssem1)
    isem = (isem0, isem1)
    cid = lax.axis_index("c")
    sid = lax.axis_index("s")
    wid = sid * NC + cid
    ebase = wid * EPW
    zv = jnp.zeros((L,), jnp.float32)

    def zu_body(i, _):
        for j in range(D_FEAT // L):
            u_v[0, i, pl.ds(j * L, L)] = zv
        return 0
    lax.fori_loop(0, CH, zu_body, 0)
    for k in range(CH // L):
        exp_v[0, pl.ds(k * L, L)] = zv
    for k in range(RPS // CH):
        pltpu.sync_copy(u_v.at[0], u_sh.at[pl.ds(sid * RPS + k * CH, CH)])
        pltpu.sync_copy(exp_v.at[0], den_sh.at[pl.ds(sid * RPS + k * CH, CH)])
    rem = RPS - (RPS // CH) * CH
    if rem:
        pltpu.sync_copy(u_v.at[0, pl.ds(0, rem)],
                        u_sh.at[pl.ds(sid * RPS + RPS - rem, rem)])
        pltpu.sync_copy(exp_v.at[0, pl.ds(0, rem)],
                        den_sh.at[pl.ds(sid * RPS + RPS - rem, rem)])
    plsc.subcore_barrier()

    pltpu.sync_copy(att_hbm, att_v)

    def idx_copy(c, q, sem, issue=True):
        s, d = sd_hbm.at[wid * NCHUNK + c], idxb.at[q]
        if issue:
            pltpu.async_copy(s, d, sem)
        else:
            pltpu.make_async_copy(s, d, sem).wait()

    def gathers(c, x, issue):
        q = c % 4
        base = pl.multiple_of(ebase + c * CH, 8)
        ops = [
            (xl_hbm.at[idxb.at[q, 0]], xl_v.at[x]),
            (xr_hbm.at[idxb.at[q, 1]], xr_v.at[x]),
            (e_hbm.at[pl.ds(base, CH)], e_v.at[x]),
        ]
        for s, d in ops:
            if issue:
                pltpu.async_copy(s, d, gsem[x])
            else:
                pltpu.make_async_copy(s, d, gsem[x]).wait()

    def scatters(c, x, issue):
        di = idxb.at[c % 4, 1]
        ops = [
            (u_v.at[x], u_sh.at[di]),
            (exp_v.at[x], den_sh.at[di]),
        ]
        for s, d in ops:
            if issue:
                pltpu.async_copy(s, d, ssem[x], add=True)
            else:
                pltpu.make_async_copy(s, d, ssem[x]).wait()

    lane = lax.iota(jnp.int32, L)
    _ib = "promise_in_bounds"

    def _allsum(v):
        for k in (8, 4, 2, 1):
            v = v + v.at[lane ^ k].get(mode=_ib)
        return v

    def compute(x):
        r = tuple(att_v[pl.ds(j * L, L)] for j in range(D_FEAT // L))

        def group_body(g, _):
            exg = jnp.zeros((L,), jnp.float32)
            for i in range(L):
                e = g * L + i
                acc = jnp.zeros((L,), jnp.float32)
                xls = []
                for j in range(D_FEAT // L):
                    xlj = xl_v[x, e, pl.ds(j * L, L)]
                    m = (xlj + xr_v[x, e, pl.ds(j * L, L)]
                         + e_v[x, e, pl.ds(j * L, L)])
                    m = jnp.maximum(m, m * 0.2)
                    acc = acc + m * r[j]
                    xls.append(xlj)
                exs = jnp.exp(_allsum(acc))
                for j in range(D_FEAT // L):
                    u_v[x, e, pl.ds(j * L, L)] = xls[j] * exs
                exg = jnp.where(lane == i, exs, exg)
            exp_v[x, pl.ds(g * L, L)] = exg
            return 0
        lax.fori_loop(0, CH // L, group_body, 0)

    idx_copy(0, 0, isem[0])
    idx_copy(0, 0, isem[0], issue=False)
    gathers(0, 0, issue=True)
    idx_copy(1, 1, isem[1])

    def pair_body(g, _):
        c = 2 * g
        for x in (0, 1):
            cc = c + x

            @pl.when(cc >= 2)
            def _():
                scatters(cc, x, issue=False)

            @pl.when(cc + 2 < NCHUNK)
            def _():
                idx_copy(cc + 2, (cc + 2) % 4, isem[x])

            @pl.when(cc + 1 < NCHUNK)
            def _():
                idx_copy(cc + 1, (cc + 1) % 4, isem[1 - x], issue=False)
                gathers(cc + 1, 1 - x, issue=True)

            gathers(cc, x, issue=False)
            compute(x)
            scatters(cc, x, issue=True)
        return 0
    lax.fori_loop(0, (NCHUNK - 1) // 2, pair_body, 0)

    ct = NCHUNK - 1
    scatters(ct, 0, issue=False)
    gathers(ct, 0, issue=False)
    compute(0)
    scatters(ct, 0, issue=True)
    scatters(ct - 1, 1, issue=False)
    scatters(ct, 0, issue=False)

    plsc.subcore_barrier()
    pltpu.sync_copy(u_sh.at[pl.ds(sid * RPS, RPS)],
                    u_out.at[pl.ds(cid * NP + sid * RPS, RPS)])
    pltpu.sync_copy(den_sh.at[pl.ds(sid * RPS, RPS)],
                    den_out.at[pl.ds(cid * NP + sid * RPS, RPS)])


_sc_edge_pass = functools.partial(
    pl.kernel,
    out_type=(jax.ShapeDtypeStruct((NC * NP, D_FEAT), jnp.float32),
              jax.ShapeDtypeStruct((NC * NP,), jnp.float32)),
    mesh=_SC_MESH,
    scratch_types=[
        pltpu.VMEM((4, 2, CH), jnp.int32),
        pltpu.VMEM((2, CH, D_FEAT), jnp.float32),
        pltpu.VMEM((2, CH, D_FEAT), jnp.float32),
        pltpu.VMEM((2, CH, D_FEAT), jnp.float32),
        pltpu.VMEM((2, CH, D_FEAT), jnp.float32),
        pltpu.VMEM((2, CH), jnp.float32),
        pltpu.VMEM((D_FEAT,), jnp.float32),
        pltpu.VMEM_SHARED((NP, D_FEAT), jnp.float32),
        pltpu.VMEM_SHARED((NP,), jnp.float32),
    ] + [pltpu.SemaphoreType.DMA] * 6,
)(_edge_body)


def _node_x(u_ref, den_ref, xl_ref, xr_ref, sa_ref, deg_ref,
            we_ref, att_ref, b_ref):
    xl = xl_ref[...]
    xr = xr_ref[...]
    u = u_ref[0] + u_ref[1]
    deg = jnp.maximum(deg_ref[0] + deg_ref[1], 1.0)
    loop_attr = (sa_ref[0] + sa_ref[1]) / deg[:, None]
    loop128 = jnp.dot(loop_attr, we_ref[...], preferred_element_type=jnp.float32)
    m = xl + xr + loop128
    m = jnp.maximum(m, m * 0.2)
    logit = jnp.sum(m * att_ref[...], axis=1)
    es = jnp.exp(logit)
    dt = den_ref[0] + den_ref[1] + es + 1e-16
    x = (u + es[:, None] * xl) / dt[:, None] + b_ref[...]
    return jnp.maximum(x, 0.0)


_ILU_SPECS = [
    pl.BlockSpec((NC, 2048, D_FEAT), lambda i: (0, i, 0)),
    pl.BlockSpec((NC, 2048), lambda i: (0, i)),
    pl.BlockSpec((2048, D_FEAT), lambda i: (i, 0)),
    pl.BlockSpec((2048, D_FEAT), lambda i: (i, 0)),
    pl.BlockSpec((NC, 2048, D_EDGE), lambda i: (0, i, 0)),
    pl.BlockSpec((NC, 2048), lambda i: (0, i)),
    pl.BlockSpec((D_EDGE, D_FEAT), lambda i: (0, 0)),
    pl.BlockSpec((1, D_FEAT), lambda i: (0, 0)),
    pl.BlockSpec((1, D_FEAT), lambda i: (0, 0)),
]


def _ilu_args(U, den, XL, XR, sa, deg, We, att, b):
    return (U.reshape(NC, NP, D_FEAT), den.reshape(NC, NP), XL, XR,
            sa.reshape(NC, NP, D_EDGE), deg.reshape(NC, NP), We,
            att.reshape(1, D_FEAT), b.reshape(1, D_FEAT))


def _interlude_mm_body(u_ref, den_ref, xl_ref, xr_ref, sa_ref, deg_ref,
                       we_ref, att_ref, b_ref, wl_ref, wr_ref,
                       xl2_ref, xr2_ref):
    x = _node_x(u_ref, den_ref, xl_ref, xr_ref, sa_ref, deg_ref,
                we_ref, att_ref, b_ref)
    xl2_ref[...] = jnp.dot(x, wl_ref[...], preferred_element_type=jnp.float32)
    xr2_ref[...] = jnp.dot(x, wr_ref[...], preferred_element_type=jnp.float32)


def _interlude_mm(U, den, XL, XR, sa, deg, We, att, b, Wl2, Wr2):
    return pl.pallas_call(
        _interlude_mm_body,
        grid=(NP // 2048,),
        in_specs=_ILU_SPECS + [
            pl.BlockSpec((D_FEAT, D_FEAT), lambda i: (0, 0)),
            pl.BlockSpec((D_FEAT, D_FEAT), lambda i: (0, 0)),
        ],
        out_specs=[
            pl.BlockSpec((2048, D_FEAT), lambda i: (i, 0)),
            pl.BlockSpec((2048, D_FEAT), lambda i: (i, 0)),
        ],
        out_shape=[
            jax.ShapeDtypeStruct((NP, D_FEAT), jnp.float32),
            jax.ShapeDtypeStruct((NP, D_FEAT), jnp.float32),
        ],
    )(*_ilu_args(U, den, XL, XR, sa, deg, We, att, b), Wl2, Wr2)


def _interlude_pool_body(u_ref, den_ref, xl_ref, xr_ref, sa_ref, deg_ref,
                         we_ref, att_ref, b_ref, batch_ref, o_ref,
                         sum_acc, cnt_acc):
    i = pl.program_id(0)
    x = _node_x(u_ref, den_ref, xl_ref, xr_ref, sa_ref, deg_ref,
                we_ref, att_ref, b_ref)
    bt = batch_ref[...]
    gids = lax.broadcasted_iota(jnp.int32, (NUM_GRAPHS, 2048), 0)
    onehot = (bt == gids).astype(jnp.float32)
    ps = jnp.dot(onehot, x, preferred_element_type=jnp.float32)
    pc = jnp.sum(onehot, axis=1)

    @pl.when(i == 0)
    def _():
        sum_acc[...] = jnp.zeros_like(sum_acc)
        cnt_acc[...] = jnp.zeros_like(cnt_acc)
    sum_acc[...] += ps
    cnt_acc[...] += pc[:, None]

    @pl.when(i == NP // 2048 - 1)
    def _():
        o_ref[...] = sum_acc[...] / jnp.maximum(cnt_acc[...], 1.0)


def _interlude_pool(U, den, XL, XR, sa, deg, We, att, b, batch):
    return pl.pallas_call(
        _interlude_pool_body,
        grid=(NP // 2048,),
        in_specs=_ILU_SPECS + [pl.BlockSpec((1, 2048), lambda i: (0, i))],
        out_specs=pl.BlockSpec((NUM_GRAPHS, D_FEAT), lambda i: (0, 0)),
        out_shape=jax.ShapeDtypeStruct((NUM_GRAPHS, D_FEAT), jnp.float32),
        scratch_shapes=[
            pltpu.VMEM((NUM_GRAPHS, D_FEAT), jnp.float32),
            pltpu.VMEM((NUM_GRAPHS, 1), jnp.float32),
        ],
    )(*_ilu_args(U, den, XL, XR, sa, deg, We, att, b),
      batch.reshape(1, NP))


def kernel(node_features, edge_index, batch, edge_attr, Wl1, Wr1, We1, att1, b1, Wl2, Wr2, We2, att2, b2):
    x0 = jnp.pad(node_features, ((0, NP - N_NODES), (0, 0)))
    npad = EPAD - N_EDGES
    src = jnp.pad(edge_index[0], (0, npad))
    dst = jnp.pad(edge_index[1], (0, npad), constant_values=NP - 1)
    ea = jnp.pad(edge_attr, ((0, npad), (0, 0)))
    batch_p = jnp.pad(batch, (0, NP - N_NODES), constant_values=NUM_GRAPHS)

    deg_p, sa_p = _sc_prepass(dst, ea)
    E1, E2 = _mm2(ea, We1, We2, 1024)

    sd = jnp.stack([src.reshape(-1, CH), dst.reshape(-1, CH)], axis=1)

    XL1, XR1 = _mm2(x0, Wl1, Wr1, 2048)
    U1, den1 = _sc_edge_pass(sd, XL1, XR1, E1, att1)
    XL2, XR2 = _interlude_mm(U1, den1, XL1, XR1, sa_p, deg_p, We1, att1, b1,
                             Wl2, Wr2)

    U2, den2 = _sc_edge_pass(sd, XL2, XR2, E2, att2)
    return _interlude_pool(U2, den2, XL2, XR2, sa_p, deg_p, We2, att2, b2,
                           batch_p)
